# baseline (speedup 1.0000x reference)
import jax
import jax.numpy as jnp
from jax import lax
from jax.experimental import pallas as pl
from jax.experimental.pallas import tpu as pltpu
from jax.experimental.pallas import tpu_sc as plsc

_N = 10000
_E = 2048
_NNZ = 320000
_C = 128
_EPS = 1e-6

_NC = 2
_NS = 16
_NW = _NC * _NS
_CH = _NNZ // _NW
_B = 125
_NB = _CH // _B

_CHP = 10240
_EB = _E + 128
_NBP = _N + 240

_mesh = plsc.VectorSubcoreMesh(core_axis_name="c", subcore_axis_name="s")


def _sc_pass(s_rows):
    if s_rows == _E:
        n_wr, rps = _NS, _E // _NS
        n_ph, qb = 1, 4
    else:
        n_wr, rps = 10, _N // 10
        n_ph, qb = 2, 2
    nbp = _NB // n_ph
    nq = nbp // qb

    def body(table, gids, sids, z128, out_acc, gid_v, sid_v, *rest):
        rows = rest[:qb]
        acc = rest[qb]
        gsem = rest[qb + 1:2 * qb + 1]
        ssem = rest[2 * qb + 1:]
        sid = lax.axis_index("s")
        cid = lax.axis_index("c")
        wid = sid * _NC + cid

        @pl.when(sid < n_wr)
        def _():
            pltpu.sync_copy(z128.at[pl.ds(0, rps)],
                            acc.at[pl.ds(sid * rps, rps)])

        plsc.subcore_barrier()

        for ph in range(n_ph):
            pltpu.sync_copy(gids.at[wid, pl.ds(ph * nbp, nbp)], gid_v)
            pltpu.sync_copy(sids.at[wid, pl.ds(ph * nbp, nbp)], sid_v)
            for p in range(qb):
                pltpu.async_copy(table.at[gid_v.at[p]], rows[p], gsem[p])

            if qb > 2:
                def step(k, carry):
                    base = qb * k
                    for p in range(qb):
                        j = base + p
                        pltpu.make_async_copy(
                            table.at[gid_v.at[j]], rows[p], gsem[p]).wait()
                        pltpu.async_copy(
                            rows[p], acc.at[sid_v.at[j]], ssem[p], add=True)
                    for p in range(qb):
                        j = base + p
                        pltpu.make_async_copy(
                            rows[p], acc.at[sid_v.at[j]], ssem[p]).wait()

                        @pl.when(k < nq - 1)
                        def _():
                            pltpu.async_copy(
                                table.at[gid_v.at[j + qb]], rows[p], gsem[p])

                    return carry
            else:
                def step(k, carry):
                    j0 = 2 * k
                    j1 = 2 * k + 1
                    pltpu.make_async_copy(
                        table.at[gid_v.at[j0]], rows[0], gsem[0]).wait()
                    pltpu.sync_copy(rows[0], acc.at[sid_v.at[j0]], add=True)
                    pltpu.make_async_copy(
                        table.at[gid_v.at[j1]], rows[1], gsem[1]).wait()

                    @pl.when(k < nq - 1)
                    def _():
                        pltpu.async_copy(
                            table.at[gid_v.at[j0 + 2]], rows[0], gsem[0])

                    pltpu.sync_copy(rows[1], acc.at[sid_v.at[j1]], add=True)

                    @pl.when(k < nq - 1)
                    def _():
                        pltpu.async_copy(
                            table.at[gid_v.at[j1 + 2]], rows[1], gsem[1])

                    return carry

            lax.fori_loop(0, nq, step, 0)
        plsc.subcore_barrier()

        @pl.when(sid < n_wr)
        def _():
            pltpu.sync_copy(acc.at[pl.ds(sid * rps, rps)],
                            out_acc.at[pl.ds(cid * s_rows + sid * rps, rps)])

    return pl.kernel(
        body,
        out_type=jax.ShapeDtypeStruct((_NC * s_rows, _C), jnp.float32),
        mesh=_mesh,
        scratch_types=([pltpu.VMEM((nbp, _B), jnp.int32),
                        pltpu.VMEM((nbp, _B), jnp.int32)]
                       + [pltpu.VMEM((_B, _C), jnp.float32)] * qb
                       + [pltpu.VMEM_SHARED((s_rows, _C), jnp.float32)]
                       + [pltpu.SemaphoreType.DMA] * (2 * qb)))


def _sc_hist():
    def body(hin, vin, out_de, out_dv, hv, vv, de_h, dv_h):
        sid = lax.axis_index("s")
        cid = lax.axis_index("c")
        wid = sid * _NC + cid
        pltpu.sync_copy(hin.at[pl.ds(wid * _CHP, _CHP)], hv)
        pltpu.sync_copy(vin.at[pl.ds(wid * _CHP, _CHP)], vv)

        z16 = jnp.zeros((16,), jnp.float32)

        def zde(i, c):
            de_h[pl.ds(i * 16, 16)] = z16
            return c

        lax.fori_loop(0, _EB // 16, zde, 0)

        def zdv(i, c):
            dv_h[pl.ds(i * 16, 16)] = z16
            return c

        lax.fori_loop(0, _NBP // 16, zdv, 0)

        ones = jnp.ones((16,), jnp.float32)

        def step(j, c):
            plsc.addupdate_scatter(de_h, [hv[pl.ds(j * 16, 16)]], ones)
            plsc.addupdate_scatter(dv_h, [vv[pl.ds(j * 16, 16)]], ones)
            return c

        lax.fori_loop(0, _CHP // 16, step, 0)

        pltpu.sync_copy(de_h, out_de.at[wid, 0])
        pltpu.sync_copy(dv_h, out_dv.at[wid, 0])

    return pl.kernel(
        body,
        out_type=(jax.ShapeDtypeStruct((_NW, 1, _EB), jnp.float32),
                  jax.ShapeDtypeStruct((_NW, 1, _NBP), jnp.float32)),
        mesh=_mesh,
        compiler_params=pltpu.CompilerParams(needs_layout_passes=False),
        scratch_types=[pltpu.VMEM((_CHP,), jnp.int32),
                       pltpu.VMEM((_CHP,), jnp.int32),
                       pltpu.VMEM((_EB,), jnp.float32),
                       pltpu.VMEM((_NBP,), jnp.float32)])


def _tc_linear(X, W, b2):
    def body(x_ref, w_ref, b_ref, o_ref):
        o_ref[...] = lax.dot_general(
            x_ref[...], w_ref[...], (((1,), (1,)), ((), ())),
            preferred_element_type=jnp.float32) + b_ref[...]

    return pl.pallas_call(
        body,
        grid=(10,),
        in_specs=[pl.BlockSpec((1000, _C), lambda i: (i, 0)),
                  pl.BlockSpec((_C, _C), lambda i: (0, 0)),
                  pl.BlockSpec((1, _C), lambda i: (0, 0))],
        out_specs=pl.BlockSpec((1000, _C), lambda i: (i, 0)),
        out_shape=jax.ShapeDtypeStruct((_N, _C), jnp.float32),
    )(X, W, b2)


def _tc_ye_finalize(ye_p, de_mat, dv_mat):
    def body(y_ref, d_ref, dv_ref, o_ref, dvc_ref):
        ones = jnp.ones((_NW, 1), jnp.float32)
        deg = lax.dot_general(
            d_ref[...], ones,
            (((0,), (0,)), ((), ())), preferred_element_type=jnp.float32)
        inv = jnp.where(deg > 0, 1.0 / jnp.maximum(deg, 1.0), 0.0)
        o_ref[...] = (y_ref[0] + y_ref[1]) * inv
        dvc_ref[...] = lax.dot_general(
            dv_ref[...], ones,
            (((0,), (0,)), ((), ())), preferred_element_type=jnp.float32)

    return pl.pallas_call(
        body,
        grid=(1,),
        in_specs=[pl.BlockSpec((2, _E, _C), lambda i: (0, 0, 0)),
                  pl.BlockSpec((_NW, _E), lambda i: (0, 0)),
                  pl.BlockSpec((_NW, _NBP), lambda i: (0, 0))],
        out_specs=[pl.BlockSpec((_E, _C), lambda i: (0, 0)),
                   pl.BlockSpec((_NBP, 1), lambda i: (0, 0))],
        out_shape=[jax.ShapeDtypeStruct((_E, _C), jnp.float32),
                   jax.ShapeDtypeStruct((_NBP, 1), jnp.float32)],
    )(ye_p, de_mat, dv_mat)


def _tc_xh(xa, att):
    def body(x_ref, a_ref, xh_ref, nx_ref):
        xh = x_ref[...] * a_ref[...]
        nx2 = jnp.sum(xh * xh, axis=1, keepdims=True)
        nx_ref[...] = jnp.sqrt(jnp.maximum(nx2, 1e-12))
        xh_ref[...] = xh.astype(jnp.bfloat16)

    return pl.pallas_call(
        body,
        grid=(10,),
        in_specs=[pl.BlockSpec((1000, _C), lambda i: (i, 0)),
                  pl.BlockSpec((1, _C), lambda i: (0, 0))],
        out_specs=[pl.BlockSpec((1000, _C), lambda i: (i, 0)),
                   pl.BlockSpec((1000, 1), lambda i: (i, 0))],
        out_shape=[jax.ShapeDtypeStruct((_N, _C), jnp.bfloat16),
                   jax.ShapeDtypeStruct((_N, 1), jnp.float32)],
    )(xa, att)


def _tc_xa_finalize(xv_p, dv_col):
    def body(x_ref, d_ref, o_ref):
        deg = d_ref[...]
        inv = jnp.where(deg > 0, 1.0 / jnp.maximum(deg, 1.0), 0.0)
        o_ref[...] = jnp.maximum((x_ref[0] + x_ref[1]) * inv, 0.0)

    return pl.pallas_call(
        body,
        grid=(10,),
        in_specs=[pl.BlockSpec((2, 1000, _C), lambda i: (0, i, 0)),
                  pl.BlockSpec((1000, 1), lambda i: (i, 0))],
        out_specs=pl.BlockSpec((1000, _C), lambda i: (i, 0)),
        out_shape=jax.ShapeDtypeStruct((_N, _C), jnp.float32),
    )(xv_p, dv_col)


def _tc_zh(ze_p, att):
    def body(z_ref, a_ref, zh_ref, nz_ref):
        zh = (z_ref[0] + z_ref[1]) * a_ref[...]
        zh_ref[...] = zh.astype(jnp.bfloat16)
        sq = lax.dot_general(
            jnp.ones((1, _C), jnp.float32), zh * zh,
            (((1,), (1,)), ((), ())), preferred_element_type=jnp.float32)
        nz_ref[...] = jnp.sqrt(jnp.maximum(sq, 1e-12))

    return pl.pallas_call(
        body,
        grid=(1,),
        in_specs=[pl.BlockSpec((2, _E, _C), lambda i: (0, 0, 0)),
                  pl.BlockSpec((1, _C), lambda i: (0, 0))],
        out_specs=[pl.BlockSpec((_E, _C), lambda i: (0, 0)),
                   pl.BlockSpec((1, _E), lambda i: (0, 0))],
        out_shape=[jax.ShapeDtypeStruct((_E, _C), jnp.bfloat16),
                   jax.ShapeDtypeStruct((1, _E), jnp.float32)],
    )(ze_p, att)


def _tc_kl(xh, nx, zh, nz):
    def body(x_ref, nx_ref, zh_ref, nz_ref, o_ref):
        i = pl.program_id(0)
        num = lax.dot_general(
            x_ref[...], zh_ref[...], (((1,), (1,)), ((), ())),
            preferred_element_type=jnp.float32)
        den = jnp.maximum(nx_ref[...] * nz_ref[...], _EPS)
        ac = jnp.clip(num / den, 0.0, 1.0)
        t1 = jnp.where(ac > 0.0, ac * jnp.log(2.0 * ac), 0.0)
        t2 = jnp.where(ac < 1.0, (1.0 - ac) * jnp.log(2.0 * (1.0 - ac)), 0.0)
        part = jnp.sum(t1 + t2, keepdims=True)

        @pl.when(i == 0)
        def _():
            o_ref[...] = jnp.zeros((1, 1), jnp.float32)

        o_ref[...] += part

    return pl.pallas_call(
        body,
        grid=(25,),
        in_specs=[pl.BlockSpec((400, _C), lambda i: (i, 0)),
                  pl.BlockSpec((400, 1), lambda i: (i, 0)),
                  pl.BlockSpec((_E, _C), lambda i: (0, 0)),
                  pl.BlockSpec((1, _E), lambda i: (0, 0))],
        out_specs=pl.BlockSpec((1, 1), lambda i: (0, 0)),
        out_shape=jax.ShapeDtypeStruct((1, 1), jnp.float32),
    )(xh, nx, zh, nz)


def _padflat(ids, pad_idx):
    x = ids.reshape(_NW, _CH)
    p = jnp.full((_NW, _CHP - _CH), pad_idx, jnp.int32)
    return jnp.concatenate([x, p], axis=1).reshape(-1)


def kernel(X, vertex_ids, hedge_ids, W, b, att):
    vids3 = vertex_ids.reshape(_NW, _NB, _B)
    hids3 = hedge_ids.reshape(_NW, _NB, _B)
    vflat = _padflat(vertex_ids, _N)
    hflat = _padflat(hedge_ids, _E)
    z128 = jnp.zeros((_N // 10, _C), jnp.float32)

    xt = _tc_linear(X, W, b.reshape(1, _C))
    de3, dv3 = _sc_hist()(hflat, vflat)

    ye_p = _sc_pass(_E)(xt, vids3, hids3, z128)
    ye, dv_col = _tc_ye_finalize(ye_p.reshape(_NC, _E, _C),
                                 de3.reshape(_NW, _EB)[:, :_E],
                                 dv3.reshape(_NW, _NBP))

    xv_p = _sc_pass(_N)(ye, hids3, vids3, z128)
    xa = _tc_xa_finalize(xv_p.reshape(_NC, _N, _C), dv_col)

    ze_p = _sc_pass(_E)(xa, vids3, hids3, z128)
    xh, nx = _tc_xh(xa, att)
    zh, nz = _tc_zh(ze_p.reshape(_NC, _E, _C), att)

    kl_sum = _tc_kl(xh, nx, zh, nz)
    kl_loss = kl_sum[0, 0] / jnp.float32(_N)
    return xa, kl_loss


# speedup vs baseline: 1.0089x; 1.0089x over previous
import jax
import jax.numpy as jnp
from jax import lax
from jax.experimental import pallas as pl
from jax.experimental.pallas import tpu as pltpu
from jax.experimental.pallas import tpu_sc as plsc

_N = 10000
_E = 2048
_NNZ = 320000
_C = 128
_EPS = 1e-6

_NC = 2
_NS = 16
_NW = _NC * _NS
_CH = _NNZ // _NW
_B = 125
_NB = _CH // _B

_CHP = 10240
_EB = _E + 128
_NBP = _N + 240

_mesh = plsc.VectorSubcoreMesh(core_axis_name="c", subcore_axis_name="s")


def _sc_pass(s_rows):
    if s_rows == _E:
        n_wr, rps = _NS, _E // _NS
        n_ph, qb = 1, 4
    else:
        n_wr, rps = 10, _N // 10
        n_ph, qb = 2, 2
    nbp = _NB // n_ph
    nq = nbp // qb

    def body(table, gids, sids, z128, out_acc, gid_v, sid_v, *rest):
        rows = rest[:qb]
        acc = rest[qb]
        gsem = rest[qb + 1:2 * qb + 1]
        ssem = rest[2 * qb + 1:]
        sid = lax.axis_index("s")
        cid = lax.axis_index("c")
        wid = sid * _NC + cid

        @pl.when(sid < n_wr)
        def _():
            pltpu.sync_copy(z128.at[pl.ds(0, rps)],
                            acc.at[pl.ds(sid * rps, rps)])

        plsc.subcore_barrier()

        for ph in range(n_ph):
            pltpu.sync_copy(gids.at[wid, pl.ds(ph * nbp, nbp)], gid_v)
            pltpu.sync_copy(sids.at[wid, pl.ds(ph * nbp, nbp)], sid_v)
            for p in range(qb):
                pltpu.async_copy(table.at[gid_v.at[p]], rows[p], gsem[p])

            if qb > 2:
                def step(k, carry):
                    base = qb * k
                    for p in range(qb):
                        j = base + p
                        pltpu.make_async_copy(
                            table.at[gid_v.at[j]], rows[p], gsem[p]).wait()
                        pltpu.async_copy(
                            rows[p], acc.at[sid_v.at[j]], ssem[p], add=True)
                    for p in range(qb):
                        j = base + p
                        pltpu.make_async_copy(
                            rows[p], acc.at[sid_v.at[j]], ssem[p]).wait()

                        @pl.when(k < nq - 1)
                        def _():
                            pltpu.async_copy(
                                table.at[gid_v.at[j + qb]], rows[p], gsem[p])

                    return carry
            else:
                def step(k, carry):
                    j0 = 2 * k
                    j1 = 2 * k + 1
                    pltpu.make_async_copy(
                        table.at[gid_v.at[j0]], rows[0], gsem[0]).wait()
                    pltpu.sync_copy(rows[0], acc.at[sid_v.at[j0]], add=True)
                    pltpu.make_async_copy(
                        table.at[gid_v.at[j1]], rows[1], gsem[1]).wait()

                    @pl.when(k < nq - 1)
                    def _():
                        pltpu.async_copy(
                            table.at[gid_v.at[j0 + 2]], rows[0], gsem[0])

                    pltpu.sync_copy(rows[1], acc.at[sid_v.at[j1]], add=True)

                    @pl.when(k < nq - 1)
                    def _():
                        pltpu.async_copy(
                            table.at[gid_v.at[j1 + 2]], rows[1], gsem[1])

                    return carry

            lax.fori_loop(0, nq, step, 0)
        plsc.subcore_barrier()

        @pl.when(sid < n_wr)
        def _():
            pltpu.sync_copy(acc.at[pl.ds(sid * rps, rps)],
                            out_acc.at[pl.ds(cid * s_rows + sid * rps, rps)])

    return pl.kernel(
        body,
        out_type=jax.ShapeDtypeStruct((_NC * s_rows, _C), jnp.float32),
        mesh=_mesh,
        scratch_types=([pltpu.VMEM((nbp, _B), jnp.int32),
                        pltpu.VMEM((nbp, _B), jnp.int32)]
                       + [pltpu.VMEM((_B, _C), jnp.float32)] * qb
                       + [pltpu.VMEM_SHARED((s_rows, _C), jnp.float32)]
                       + [pltpu.SemaphoreType.DMA] * (2 * qb)))


def _sc_hist():
    def body(hin, vin, out_de, out_dv, hv, vv, de_h, dv_h):
        sid = lax.axis_index("s")
        cid = lax.axis_index("c")
        wid = sid * _NC + cid
        pltpu.sync_copy(hin.at[pl.ds(wid * _CHP, _CHP)], hv)
        pltpu.sync_copy(vin.at[pl.ds(wid * _CHP, _CHP)], vv)

        z16 = jnp.zeros((16,), jnp.float32)

        def zde(i, c):
            de_h[pl.ds(i * 16, 16)] = z16
            return c

        lax.fori_loop(0, _EB // 16, zde, 0)

        def zdv(i, c):
            dv_h[pl.ds(i * 16, 16)] = z16
            return c

        lax.fori_loop(0, _NBP // 16, zdv, 0)

        ones = jnp.ones((16,), jnp.float32)

        def step(j, c):
            plsc.addupdate_scatter(de_h, [hv[pl.ds(j * 16, 16)]], ones)
            plsc.addupdate_scatter(dv_h, [vv[pl.ds(j * 16, 16)]], ones)
            return c

        lax.fori_loop(0, _CHP // 16, step, 0)

        pltpu.sync_copy(de_h, out_de.at[wid, 0])
        pltpu.sync_copy(dv_h, out_dv.at[wid, 0])

    return pl.kernel(
        body,
        out_type=(jax.ShapeDtypeStruct((_NW, 1, _EB), jnp.float32),
                  jax.ShapeDtypeStruct((_NW, 1, _NBP), jnp.float32)),
        mesh=_mesh,
        compiler_params=pltpu.CompilerParams(needs_layout_passes=False),
        scratch_types=[pltpu.VMEM((_CHP,), jnp.int32),
                       pltpu.VMEM((_CHP,), jnp.int32),
                       pltpu.VMEM((_EB,), jnp.float32),
                       pltpu.VMEM((_NBP,), jnp.float32)])


def _tc_linear(X, W, b2):
    def body(x_ref, w_ref, b_ref, o_ref):
        o_ref[...] = lax.dot_general(
            x_ref[...], w_ref[...], (((1,), (1,)), ((), ())),
            preferred_element_type=jnp.float32) + b_ref[...]

    return pl.pallas_call(
        body,
        grid=(10,),
        in_specs=[pl.BlockSpec((1000, _C), lambda i: (i, 0)),
                  pl.BlockSpec((_C, _C), lambda i: (0, 0)),
                  pl.BlockSpec((1, _C), lambda i: (0, 0))],
        out_specs=pl.BlockSpec((1000, _C), lambda i: (i, 0)),
        out_shape=jax.ShapeDtypeStruct((_N, _C), jnp.float32),
    )(X, W, b2)


def _tc_ye_finalize(ye_p, de_mat, dv_mat):
    def body(y_ref, d_ref, dv_ref, o_ref, dvc_ref):
        ones = jnp.ones((_NW, 1), jnp.float32)
        deg = lax.dot_general(
            d_ref[...], ones,
            (((0,), (0,)), ((), ())), preferred_element_type=jnp.float32)
        inv = jnp.where(deg > 0, 1.0 / jnp.maximum(deg, 1.0), 0.0)
        o_ref[...] = (y_ref[0] + y_ref[1]) * inv
        dvc_ref[...] = lax.dot_general(
            dv_ref[...], ones,
            (((0,), (0,)), ((), ())), preferred_element_type=jnp.float32)

    return pl.pallas_call(
        body,
        grid=(1,),
        in_specs=[pl.BlockSpec((2, _E, _C), lambda i: (0, 0, 0)),
                  pl.BlockSpec((_NW, _E), lambda i: (0, 0)),
                  pl.BlockSpec((_NW, _NBP), lambda i: (0, 0))],
        out_specs=[pl.BlockSpec((_E, _C), lambda i: (0, 0)),
                   pl.BlockSpec((_NBP, 1), lambda i: (0, 0))],
        out_shape=[jax.ShapeDtypeStruct((_E, _C), jnp.float32),
                   jax.ShapeDtypeStruct((_NBP, 1), jnp.float32)],
    )(ye_p, de_mat, dv_mat)


def _tc_xh(xa, att):
    def body(x_ref, a_ref, xh_ref, nx_ref):
        xh = x_ref[...] * a_ref[...]
        nx2 = jnp.sum(xh * xh, axis=1, keepdims=True)
        nx_ref[...] = jnp.sqrt(jnp.maximum(nx2, 1e-12))
        xh_ref[...] = xh.astype(jnp.bfloat16)

    return pl.pallas_call(
        body,
        grid=(10,),
        in_specs=[pl.BlockSpec((1000, _C), lambda i: (i, 0)),
                  pl.BlockSpec((1, _C), lambda i: (0, 0))],
        out_specs=[pl.BlockSpec((1000, _C), lambda i: (i, 0)),
                   pl.BlockSpec((1000, 1), lambda i: (i, 0))],
        out_shape=[jax.ShapeDtypeStruct((_N, _C), jnp.bfloat16),
                   jax.ShapeDtypeStruct((_N, 1), jnp.float32)],
    )(xa, att)


def _tc_xa_finalize(xv_p, dv_col):
    def body(x_ref, d_ref, o_ref):
        deg = d_ref[...]
        inv = jnp.where(deg > 0, 1.0 / jnp.maximum(deg, 1.0), 0.0)
        o_ref[...] = jnp.maximum((x_ref[0] + x_ref[1]) * inv, 0.0)

    return pl.pallas_call(
        body,
        grid=(10,),
        in_specs=[pl.BlockSpec((2, 1000, _C), lambda i: (0, i, 0)),
                  pl.BlockSpec((1000, 1), lambda i: (i, 0))],
        out_specs=pl.BlockSpec((1000, _C), lambda i: (i, 0)),
        out_shape=jax.ShapeDtypeStruct((_N, _C), jnp.float32),
    )(xv_p, dv_col)


def _tc_zh(ze_p, att):
    def body(z_ref, a_ref, zh_ref, nz_ref):
        zh = (z_ref[0] + z_ref[1]) * a_ref[...]
        zh_ref[...] = zh.astype(jnp.bfloat16)
        sq = lax.dot_general(
            jnp.ones((1, _C), jnp.float32), zh * zh,
            (((1,), (1,)), ((), ())), preferred_element_type=jnp.float32)
        nz_ref[...] = jnp.sqrt(jnp.maximum(sq, 1e-12))

    return pl.pallas_call(
        body,
        grid=(1,),
        in_specs=[pl.BlockSpec((2, _E, _C), lambda i: (0, 0, 0)),
                  pl.BlockSpec((1, _C), lambda i: (0, 0))],
        out_specs=[pl.BlockSpec((_E, _C), lambda i: (0, 0)),
                   pl.BlockSpec((1, _E), lambda i: (0, 0))],
        out_shape=[jax.ShapeDtypeStruct((_E, _C), jnp.bfloat16),
                   jax.ShapeDtypeStruct((1, _E), jnp.float32)],
    )(ze_p, att)


def _tc_kl(xh, nx, zh, nz):
    def body(x_ref, nx_ref, zh_ref, nz_ref, o_ref):
        i = pl.program_id(0)
        num = lax.dot_general(
            x_ref[...], zh_ref[...], (((1,), (1,)), ((), ())),
            preferred_element_type=jnp.float32)
        den = jnp.maximum(nx_ref[...] * nz_ref[...], _EPS)
        ac = jnp.clip(num / den, 0.0, 1.0)
        t1 = jnp.where(ac > 0.0, ac * jnp.log(2.0 * ac), 0.0)
        t2 = jnp.where(ac < 1.0, (1.0 - ac) * jnp.log(2.0 * (1.0 - ac)), 0.0)
        part = jnp.sum(t1 + t2, keepdims=True)

        @pl.when(i == 0)
        def _():
            o_ref[...] = jnp.zeros((1, 1), jnp.float32)

        o_ref[...] += part

    return pl.pallas_call(
        body,
        grid=(10,),
        in_specs=[pl.BlockSpec((1000, _C), lambda i: (i, 0)),
                  pl.BlockSpec((1000, 1), lambda i: (i, 0)),
                  pl.BlockSpec((_E, _C), lambda i: (0, 0)),
                  pl.BlockSpec((1, _E), lambda i: (0, 0))],
        out_specs=pl.BlockSpec((1, 1), lambda i: (0, 0)),
        out_shape=jax.ShapeDtypeStruct((1, 1), jnp.float32),
    )(xh, nx, zh, nz)


def _padflat(ids, pad_idx):
    x = ids.reshape(_NW, _CH)
    p = jnp.full((_NW, _CHP - _CH), pad_idx, jnp.int32)
    return jnp.concatenate([x, p], axis=1).reshape(-1)


def kernel(X, vertex_ids, hedge_ids, W, b, att):
    vids3 = vertex_ids.reshape(_NW, _NB, _B)
    hids3 = hedge_ids.reshape(_NW, _NB, _B)
    vflat = _padflat(vertex_ids, _N)
    hflat = _padflat(hedge_ids, _E)
    z128 = jnp.zeros((_N // 10, _C), jnp.float32)

    xt = _tc_linear(X, W, b.reshape(1, _C))
    de3, dv3 = _sc_hist()(hflat, vflat)

    ye_p = _sc_pass(_E)(xt, vids3, hids3, z128)
    ye, dv_col = _tc_ye_finalize(ye_p.reshape(_NC, _E, _C),
                                 de3.reshape(_NW, _EB)[:, :_E],
                                 dv3.reshape(_NW, _NBP))

    xv_p = _sc_pass(_N)(ye, hids3, vids3, z128)
    xa = _tc_xa_finalize(xv_p.reshape(_NC, _N, _C), dv_col)

    ze_p = _sc_pass(_E)(xa, vids3, hids3, z128)
    xh, nx = _tc_xh(xa, att)
    zh, nz = _tc_zh(ze_p.reshape(_NC, _E, _C), att)

    kl_sum = _tc_kl(xh, nx, zh, nz)
    kl_loss = kl_sum[0, 0] / jnp.float32(_N)
    return xa, kl_loss
